# Initial kernel scaffold; baseline (speedup 1.0000x reference)
import jax
import jax.numpy as jnp
from jax.experimental import pallas as pl


def kernel(x):
    raise NotImplementedError("write your pallas kernel here")


# speedup vs baseline: 28.3391x; 28.3391x over previous
import jax
import jax.numpy as jnp
from jax.experimental import pallas as pl

_N = 8192
_BLOCK_ROWS = 128
_K = 4095


def _select_body(x_ref, o_ref):
    xb = x_ref[...]
    bits = jax.lax.bitcast_convert_type(xb, jnp.int32)
    keys = bits ^ (jax.lax.shift_right_arithmetic(bits, 31) & jnp.int32(0x7FFFFFFF))
    lo = jnp.full((_BLOCK_ROWS, 1), jnp.int32(-(2**31)))
    k = jnp.full((_BLOCK_ROWS, 1), _K, jnp.int32)
    for b in range(31, -1, -1):
        mid = lo + (jnp.int32(1) << jnp.int32(b))
        c = jnp.sum(
            jnp.logical_and(keys >= lo, keys < mid).astype(jnp.int32),
            axis=1,
            keepdims=True,
        )
        go_hi = k >= c
        lo = jnp.where(go_hi, mid, lo)
        k = jnp.where(go_hi, k - c, k)
    key_sel = lo[:, 0]
    out_bits = key_sel ^ (
        jax.lax.shift_right_arithmetic(key_sel, 31) & jnp.int32(0x7FFFFFFF)
    )
    o_ref[0, 0, :] = jax.lax.bitcast_convert_type(out_bits, jnp.float32)


def kernel(x):
    b, h, n = x.shape
    rows = b * h
    xf = x.reshape(rows, n)
    grid = rows // _BLOCK_ROWS
    out = pl.pallas_call(
        _select_body,
        grid=(grid,),
        in_specs=[pl.BlockSpec((_BLOCK_ROWS, n), lambda i: (i, 0))],
        out_specs=pl.BlockSpec((1, 1, _BLOCK_ROWS), lambda i: (i, 0, 0)),
        out_shape=jax.ShapeDtypeStruct((grid, 1, _BLOCK_ROWS), jnp.float32),
    )(xf)
    return out.reshape(b, h)
